# baseline (speedup 1.0000x reference)
import functools

import jax
import jax.numpy as jnp
from jax import lax
from jax.experimental import pallas as pl
from jax.experimental.pallas import tpu as pltpu
from jax.experimental.pallas import tpu_sc as plsc

N = 10000
E = 320000
D = 128
OUT = 128
BN_EPS = 1e-5

NC = 2
NS = 16
NW = NC * NS
CHUNK = 128
CH_PER_W = 80
E_PAD = NW * CH_PER_W * CHUNK
N_PAD = 10240
ROWS_PER_TILE = N_PAD // NS


def _sc_aggregate(feat, srcp, dstp):
    mesh = plsc.VectorSubcoreMesh(core_axis_name="c", subcore_axis_name="s")

    @functools.partial(
        pl.kernel,
        mesh=mesh,
        out_type=jax.ShapeDtypeStruct((NC, N_PAD, D), jnp.float32),
        scratch_types=[
            pltpu.VMEM((2, 1, CHUNK), jnp.int32),
            pltpu.VMEM((2, 1, CHUNK), jnp.int32),
            pltpu.VMEM((2, 1, CHUNK), jnp.int32),
            pltpu.VMEM((2, 1, CHUNK), jnp.int32),
            pltpu.VMEM((CHUNK, D), jnp.float32),
            pltpu.VMEM((CHUNK, D), jnp.float32),
            pltpu.VMEM_SHARED((N_PAD, D), jnp.float32),
            pltpu.SemaphoreType.DMA,
            pltpu.SemaphoreType.DMA,
            pltpu.SemaphoreType.DMA,
            pltpu.SemaphoreType.DMA,
            pltpu.SemaphoreType.DMA,
            pltpu.SemaphoreType.DMA,
        ],
    )
    def k(feat_hbm, src_hbm, dst_hbm, out_hbm,
          idx0, idx1, idx2, idx3, rows0, rows1, acc,
          isem0, isem1, isem2, isem3, gsem0, gsem1):
        idxs = (idx0, idx1, idx2, idx3)
        isems = (isem0, isem1, isem2, isem3)
        rows = (rows0, rows1)
        gsems = (gsem0, gsem1)
        cid = lax.axis_index("c")
        sid = lax.axis_index("s")
        wid = sid * NC + cid

        @pl.when(cid == 0)
        def _():
            pltpu.sync_copy(feat_hbm.at[pl.ds(sid * ROWS_PER_TILE, ROWS_PER_TILE)],
                            acc.at[pl.ds(sid * ROWS_PER_TILE, ROWS_PER_TILE)])

        @pl.when(cid == 1)
        def _():
            for t, sz in ((0, 240), (240, 240), (480, 160)):
                pltpu.sync_copy(
                    feat_hbm.at[pl.ds(N, sz)],
                    acc.at[pl.ds(sid * ROWS_PER_TILE + t, sz)])

        for b, buf in ((0, idx0), (1, idx1)):
            pltpu.sync_copy(src_hbm.at[pl.ds(b * NW + wid, 1)], buf.at[pl.ds(0, 1)])
            pltpu.sync_copy(dst_hbm.at[pl.ds(b * NW + wid, 1)], buf.at[pl.ds(1, 1)])
        pltpu.async_copy(src_hbm.at[pl.ds(2 * NW + wid, 1)], idx2.at[pl.ds(0, 1)], isem2)
        pltpu.async_copy(dst_hbm.at[pl.ds(2 * NW + wid, 1)], idx2.at[pl.ds(1, 1)], isem2)
        pltpu.async_copy(src_hbm.at[pl.ds(3 * NW + wid, 1)], idx3.at[pl.ds(0, 1)], isem3)
        pltpu.async_copy(dst_hbm.at[pl.ds(3 * NW + wid, 1)], idx3.at[pl.ds(1, 1)], isem3)
        plsc.subcore_barrier()
        pltpu.async_copy(feat_hbm.at[idx0.at[0, 0]], rows0, gsem0)
        pltpu.async_copy(feat_hbm.at[idx1.at[0, 0]], rows1, gsem1)

        @pl.loop(0, CH_PER_W, step=4)
        def _(j):
            for b in range(4):
                c = j + b
                rb, gs = rows[b % 2], gsems[b % 2]
                pltpu.make_async_copy(feat_hbm.at[idxs[b].at[0, 0]],
                                      rb, gs).wait()
                pltpu.sync_copy(rb, acc.at[idxs[b].at[1, 0]], add=True)

                @pl.when(c + 4 < CH_PER_W)
                def _():
                    pltpu.async_copy(src_hbm.at[pl.ds((c + 4) * NW + wid, 1)],
                                     idxs[b].at[pl.ds(0, 1)], isems[b])
                    pltpu.async_copy(dst_hbm.at[pl.ds((c + 4) * NW + wid, 1)],
                                     idxs[b].at[pl.ds(1, 1)], isems[b])

                @pl.when(c + 2 < CH_PER_W)
                def _():
                    b2 = (b + 2) % 4
                    pltpu.make_async_copy(
                        src_hbm.at[pl.ds((c + 2) * NW + wid, 1)],
                        idxs[b2].at[pl.ds(0, 1)], isems[b2]).wait()
                    pltpu.make_async_copy(
                        dst_hbm.at[pl.ds((c + 2) * NW + wid, 1)],
                        idxs[b2].at[pl.ds(1, 1)], isems[b2]).wait()
                    pltpu.async_copy(feat_hbm.at[idxs[b2].at[0, 0]], rb, gs)

        plsc.subcore_barrier()
        pltpu.sync_copy(acc.at[pl.ds(sid * ROWS_PER_TILE, ROWS_PER_TILE)],
                        out_hbm.at[cid, pl.ds(sid * ROWS_PER_TILE, ROWS_PER_TILE)])

    return k(feat, srcp, dstp)


def _tc_layer1(p, W1a, b1a, W1b, b1b, g1, be1):
    def body(p_ref, wa_ref, ba_ref, wb_ref, bb_ref, g_ref, be_ref, o_ref):
        agg = p_ref[0, :N, :] + p_ref[1, :N, :]
        t = jnp.dot(agg.astype(jnp.bfloat16), wa_ref[...].astype(jnp.bfloat16),
                    preferred_element_type=jnp.float32)
        t = jnp.maximum(t + ba_ref[...], 0.0)
        h = jnp.dot(t.astype(jnp.bfloat16), wb_ref[...].astype(jnp.bfloat16),
                    preferred_element_type=jnp.float32)
        h = h + bb_ref[...]
        mean = jnp.mean(h, axis=0, keepdims=True)
        var = jnp.mean((h - mean) ** 2, axis=0, keepdims=True)
        h = (h - mean) * lax.rsqrt(var + BN_EPS) * g_ref[...] + be_ref[...]
        o_ref[0:N, :] = jnp.maximum(h, 0.0)
        o_ref[N:N_PAD, :] = jnp.zeros((N_PAD - N, D), jnp.float32)

    return pl.pallas_call(
        body,
        out_shape=jax.ShapeDtypeStruct((N_PAD, D), jnp.float32),
    )(p, W1a, b1a.reshape(1, D), W1b, b1b.reshape(1, D),
      g1.reshape(1, D), be1.reshape(1, D))


def _tc_layer2(q, W2a, b2a, W2b, b2b, g2, be2, Wf, bf):
    def body(p_ref, wa_ref, ba_ref, wb_ref, bb_ref, g_ref, be_ref,
             wf_ref, bf_ref, o_ref):
        agg = p_ref[0, :N, :] + p_ref[1, :N, :]
        t = jnp.dot(agg.astype(jnp.bfloat16), wa_ref[...].astype(jnp.bfloat16),
                    preferred_element_type=jnp.float32)
        t = jnp.maximum(t + ba_ref[...], 0.0)
        h = jnp.dot(t.astype(jnp.bfloat16), wb_ref[...].astype(jnp.bfloat16),
                    preferred_element_type=jnp.float32)
        h = h + bb_ref[...]
        mean = jnp.mean(h, axis=0, keepdims=True)
        var = jnp.mean((h - mean) ** 2, axis=0, keepdims=True)
        h = (h - mean) * lax.rsqrt(var + BN_EPS) * g_ref[...] + be_ref[...]
        o_ref[...] = jnp.dot(h.astype(jnp.bfloat16), wf_ref[...].astype(jnp.bfloat16),
                             preferred_element_type=jnp.float32) + bf_ref[...]

    return pl.pallas_call(
        body,
        out_shape=jax.ShapeDtypeStruct((N, OUT), jnp.float32),
    )(q, W2a, b2a.reshape(1, D), W2b, b2b.reshape(1, D),
      g2.reshape(1, D), be2.reshape(1, D), Wf, bf.reshape(1, OUT))


def kernel(x, edge_index, W1a, b1a, W1b, b1b, g1, be1,
           W2a, b2a, W2b, b2b, g2, be2, Wf, bf):
    src = edge_index[0].astype(jnp.int32)
    dst = edge_index[1].astype(jnp.int32)
    npad = E_PAD - E
    pad_src = jnp.arange(npad, dtype=jnp.int32) % N
    srcp = jnp.concatenate([src, pad_src])
    pad_dst = N + (jnp.arange(npad, dtype=jnp.int32) % (N_PAD - N))
    dstp = jnp.concatenate([dst, pad_dst])
    srcp = srcp.reshape(NW * CH_PER_W, 1, CHUNK)
    dstp = dstp.reshape(NW * CH_PER_W, 1, CHUNK)

    xpad = jnp.concatenate([x, jnp.zeros((N_PAD - N, D), jnp.float32)])
    p = _sc_aggregate(xpad, srcp, dstp)
    h1 = _tc_layer1(p, W1a, b1a, W1b, b1b, g1, be1)
    q = _sc_aggregate(h1, srcp, dstp)
    return _tc_layer2(q, W2a, b2a, W2b, b2b, g2, be2, Wf, bf)


# speedup vs baseline: 1.0398x; 1.0398x over previous
import functools

import jax
import jax.numpy as jnp
from jax import lax
from jax.experimental import pallas as pl
from jax.experimental.pallas import tpu as pltpu
from jax.experimental.pallas import tpu_sc as plsc

N = 10000
E = 320000
D = 128
OUT = 128
BN_EPS = 1e-5

NC = 2
NS = 16
NW = NC * NS
CHUNK = 128
CH_PER_W = 80
NCH = NW * CH_PER_W
NCH_REAL = E // CHUNK
NCH_PAD = NCH - NCH_REAL
N_PAD = 10240
ROWS_PER_TILE = N_PAD // NS
ZROWS = 320


def _sc_aggregate(feat, src2, dst2, psrc, pdst, zc):
    mesh = plsc.VectorSubcoreMesh(core_axis_name="c", subcore_axis_name="s")

    @functools.partial(
        pl.kernel,
        mesh=mesh,
        out_type=jax.ShapeDtypeStruct((NC, N_PAD, D), jnp.float32),
        scratch_types=[
            pltpu.VMEM((2, 1, CHUNK), jnp.int32),
            pltpu.VMEM((2, 1, CHUNK), jnp.int32),
            pltpu.VMEM((2, 1, CHUNK), jnp.int32),
            pltpu.VMEM((2, 1, CHUNK), jnp.int32),
            pltpu.VMEM((CHUNK, D), jnp.float32),
            pltpu.VMEM((CHUNK, D), jnp.float32),
            pltpu.VMEM_SHARED((N_PAD, D), jnp.float32),
            pltpu.SemaphoreType.DMA,
            pltpu.SemaphoreType.DMA,
            pltpu.SemaphoreType.DMA,
            pltpu.SemaphoreType.DMA,
            pltpu.SemaphoreType.DMA,
            pltpu.SemaphoreType.DMA,
        ],
    )
    def k(feat_hbm, src_hbm, dst_hbm, psrc_hbm, pdst_hbm, z_hbm, out_hbm,
          idx0, idx1, idx2, idx3, rows0, rows1, acc,
          isem0, isem1, isem2, isem3, gsem0, gsem1):
        idxs = (idx0, idx1, idx2, idx3)
        isems = (isem0, isem1, isem2, isem3)
        rows = (rows0, rows1)
        gsems = (gsem0, gsem1)
        cid = lax.axis_index("c")
        sid = lax.axis_index("s")
        wid = sid * NC + cid
        row0 = sid * ROWS_PER_TILE

        def idx_fetch(r, buf, sem, sync):
            copy = pltpu.sync_copy if sync else pltpu.async_copy

            @pl.when(r < NCH_REAL)
            def _():
                args = () if sync else (sem,)
                copy(src_hbm.at[pl.ds(r, 1)], buf.at[pl.ds(0, 1)], *args)
                copy(dst_hbm.at[pl.ds(r, 1)], buf.at[pl.ds(1, 1)], *args)

            @pl.when(r >= NCH_REAL)
            def _():
                args = () if sync else (sem,)
                copy(psrc_hbm.at[pl.ds(r - NCH_REAL, 1)], buf.at[pl.ds(0, 1)], *args)
                copy(pdst_hbm.at[pl.ds(r - NCH_REAL, 1)], buf.at[pl.ds(1, 1)], *args)

        def idx_wait(buf, sem):
            pltpu.make_async_copy(src_hbm.at[pl.ds(0, 1)],
                                  buf.at[pl.ds(0, 1)], sem).wait()
            pltpu.make_async_copy(src_hbm.at[pl.ds(0, 1)],
                                  buf.at[pl.ds(1, 1)], sem).wait()

        @pl.when(jnp.logical_and(cid == 0, sid < NS - 1))
        def _():
            pltpu.sync_copy(feat_hbm.at[pl.ds(row0, ROWS_PER_TILE)],
                            acc.at[pl.ds(row0, ROWS_PER_TILE)])

        @pl.when(jnp.logical_and(cid == 0, sid == NS - 1))
        def _():
            pltpu.sync_copy(feat_hbm.at[pl.ds(N - ROWS_PER_TILE + (N_PAD - N),
                                              ROWS_PER_TILE - (N_PAD - N))],
                            acc.at[pl.ds(row0, ROWS_PER_TILE - (N_PAD - N))])

        @pl.when(jnp.logical_and(cid == 1, sid < NS - 1))
        def _():
            pltpu.sync_copy(z_hbm, acc.at[pl.ds(row0, ZROWS)])
            pltpu.sync_copy(z_hbm, acc.at[pl.ds(row0 + ZROWS, ZROWS)])

        @pl.when(jnp.logical_and(cid == 1, sid == NS - 1))
        def _():
            pltpu.sync_copy(z_hbm, acc.at[pl.ds(row0, ZROWS)])
            pltpu.sync_copy(z_hbm.at[pl.ds(0, N - ROWS_PER_TILE * (NS - 1) - ZROWS)],
                            acc.at[pl.ds(row0 + ZROWS,
                                         N - ROWS_PER_TILE * (NS - 1) - ZROWS)])

        idx_fetch(0 * NW + wid, idx0, None, sync=True)
        idx_fetch(1 * NW + wid, idx1, None, sync=True)
        idx_fetch(2 * NW + wid, idx2, isem2, sync=False)
        idx_fetch(3 * NW + wid, idx3, isem3, sync=False)
        plsc.subcore_barrier()
        pltpu.async_copy(feat_hbm.at[idx0.at[0, 0]], rows0, gsem0)
        pltpu.async_copy(feat_hbm.at[idx1.at[0, 0]], rows1, gsem1)

        @pl.loop(0, CH_PER_W, step=4)
        def _(j):
            for b in range(4):
                c = j + b
                rb, gs = rows[b % 2], gsems[b % 2]
                pltpu.make_async_copy(feat_hbm.at[idxs[b].at[0, 0]],
                                      rb, gs).wait()
                pltpu.sync_copy(rb, acc.at[idxs[b].at[1, 0]], add=True)

                @pl.when(c + 4 < CH_PER_W)
                def _():
                    idx_fetch((c + 4) * NW + wid, idxs[b], isems[b], sync=False)

                @pl.when(c + 2 < CH_PER_W)
                def _():
                    b2 = (b + 2) % 4
                    idx_wait(idxs[b2], isems[b2])
                    pltpu.async_copy(feat_hbm.at[idxs[b2].at[0, 0]], rb, gs)

        plsc.subcore_barrier()
        pltpu.sync_copy(acc.at[pl.ds(row0, ROWS_PER_TILE)],
                        out_hbm.at[cid, pl.ds(row0, ROWS_PER_TILE)])

    return k(feat, src2, dst2, psrc, pdst, zc)


def _tc_layer1(p, W1a, b1a, W1b, b1b, g1, be1):
    def body(p_ref, wa_ref, ba_ref, wb_ref, bb_ref, g_ref, be_ref, o_ref):
        agg = p_ref[0, :N, :] + p_ref[1, :N, :]
        t = jnp.dot(agg.astype(jnp.bfloat16), wa_ref[...].astype(jnp.bfloat16),
                    preferred_element_type=jnp.float32)
        t = jnp.maximum(t + ba_ref[...], 0.0)
        h = jnp.dot(t.astype(jnp.bfloat16), wb_ref[...].astype(jnp.bfloat16),
                    preferred_element_type=jnp.float32)
        h = h + bb_ref[...]
        mean = jnp.mean(h, axis=0, keepdims=True)
        var = jnp.mean((h - mean) ** 2, axis=0, keepdims=True)
        h = (h - mean) * lax.rsqrt(var + BN_EPS) * g_ref[...] + be_ref[...]
        o_ref[...] = jnp.maximum(h, 0.0)

    return pl.pallas_call(
        body,
        out_shape=jax.ShapeDtypeStruct((N, D), jnp.float32),
    )(p, W1a, b1a.reshape(1, D), W1b, b1b.reshape(1, D),
      g1.reshape(1, D), be1.reshape(1, D))


def _tc_layer2(q, W2a, b2a, W2b, b2b, g2, be2, Wf, bf):
    def body(p_ref, wa_ref, ba_ref, wb_ref, bb_ref, g_ref, be_ref,
             wf_ref, bf_ref, o_ref):
        agg = p_ref[0, :N, :] + p_ref[1, :N, :]
        t = jnp.dot(agg.astype(jnp.bfloat16), wa_ref[...].astype(jnp.bfloat16),
                    preferred_element_type=jnp.float32)
        t = jnp.maximum(t + ba_ref[...], 0.0)
        h = jnp.dot(t.astype(jnp.bfloat16), wb_ref[...].astype(jnp.bfloat16),
                    preferred_element_type=jnp.float32)
        h = h + bb_ref[...]
        mean = jnp.mean(h, axis=0, keepdims=True)
        var = jnp.mean((h - mean) ** 2, axis=0, keepdims=True)
        h = (h - mean) * lax.rsqrt(var + BN_EPS) * g_ref[...] + be_ref[...]
        o_ref[...] = jnp.dot(h.astype(jnp.bfloat16), wf_ref[...].astype(jnp.bfloat16),
                             preferred_element_type=jnp.float32) + bf_ref[...]

    return pl.pallas_call(
        body,
        out_shape=jax.ShapeDtypeStruct((N, OUT), jnp.float32),
    )(q, W2a, b2a.reshape(1, D), W2b, b2b.reshape(1, D),
      g2.reshape(1, D), be2.reshape(1, D), Wf, bf.reshape(1, OUT))


def kernel(x, edge_index, W1a, b1a, W1b, b1b, g1, be1,
           W2a, b2a, W2b, b2b, g2, be2, Wf, bf):
    ei = edge_index.astype(jnp.int32)
    src2 = ei[0].reshape(NCH_REAL, 1, CHUNK)
    dst2 = ei[1].reshape(NCH_REAL, 1, CHUNK)
    npad = NCH_PAD * CHUNK
    psrc = (jnp.arange(npad, dtype=jnp.int32) % N).reshape(NCH_PAD, 1, CHUNK)
    pdst = (N + jnp.arange(npad, dtype=jnp.int32) % (N_PAD - N)).reshape(
        NCH_PAD, 1, CHUNK)
    zc = jnp.zeros((ZROWS, D), jnp.float32)

    p = _sc_aggregate(x, src2, dst2, psrc, pdst, zc)
    h1 = _tc_layer1(p, W1a, b1a, W1b, b1b, g1, be1)
    q = _sc_aggregate(h1, src2, dst2, psrc, pdst, zc)
    return _tc_layer2(q, W2a, b2a, W2b, b2b, g2, be2, Wf, bf)
